# Initial kernel scaffold; baseline (speedup 1.0000x reference)
import functools

import jax
import jax.numpy as jnp
from jax import lax
from jax.experimental import pallas as pl
from jax.experimental.pallas import tpu as pltpu
from jax.experimental.pallas import tpu_sc as plsc

N = 10000
D = 256
H = 128
E = 160000
NC = 2
NS = 16
B = 128
NBLK = 80
EPT = B * NBLK
EPAD = EPT * NS
NPT = 640
NPAD = NPT * NS
RB = 128
NRB = NPT // RB


def _sc_body(tab_ui, tab_iu, sidx_ui, didx_ui, sidx_iu, didx_iu,
             mean_out,
             sidx_v, didx_v, rows_v, ones_v, zero_v, zero16_v, norm_v, cntb_v,
             acc_sh, cnt_sh, gsem):
    c = lax.axis_index("c")
    s = lax.axis_index("s")
    ebase = s * EPT
    rbase = s * NPT

    def _fill(r, carry):
        zero16_v[r, :] = jnp.zeros((16,), jnp.float32)
        ones_v[r, :] = jnp.full((16,), 1.0, jnp.float32)
        for j in range(H // 16):
            zero_v[r, pl.ds(j * 16, 16)] = jnp.zeros((16,), jnp.float32)
        return carry

    lax.fori_loop(0, RB, _fill, 0)

    for d, (tab, sidx, didx) in enumerate(
        ((tab_ui, sidx_ui, didx_ui), (tab_iu, sidx_iu, didx_iu))
    ):
        for k in range(NRB):
            pltpu.sync_copy(zero_v, acc_sh.at[pl.ds(rbase + k * RB, RB)])
            pltpu.sync_copy(zero16_v, cnt_sh.at[pl.ds(rbase + k * RB, RB)])
        plsc.subcore_barrier()

        def _blk(b, carry):
            off = ebase + b * B
            pltpu.sync_copy(sidx.at[c, pl.ds(off, B)], sidx_v)
            pltpu.sync_copy(didx.at[pl.ds(off, B)], didx_v)
            pltpu.async_copy(tab.at[sidx_v], rows_v, gsem).wait()
            pltpu.sync_copy(rows_v, acc_sh.at[didx_v], add=True)
            pltpu.sync_copy(ones_v, cnt_sh.at[didx_v], add=True)
            return carry

        lax.fori_loop(0, NBLK, _blk, 0)
        plsc.subcore_barrier()

        for k in range(NRB):
            r0 = rbase + k * RB
            pltpu.sync_copy(acc_sh.at[pl.ds(r0, RB)], norm_v)
            pltpu.sync_copy(cnt_sh.at[pl.ds(r0, RB)], cntb_v)

            def _norm(r, carry):
                scale = 1.0 / jnp.maximum(cntb_v[r, :], 1.0)
                for j in range(H // 16):
                    norm_v[r, pl.ds(j * 16, 16)] = (
                        norm_v[r, pl.ds(j * 16, 16)] * scale
                    )
                return carry

            lax.fori_loop(0, RB, _norm, 0)
            pltpu.sync_copy(norm_v, mean_out.at[d, c, pl.ds(r0, RB)])
        plsc.subcore_barrier()


_sc_agg = functools.partial(
    pl.kernel,
    out_type=jax.ShapeDtypeStruct((2, NC, NPAD, H), jnp.float32),
    mesh=plsc.VectorSubcoreMesh(
        core_axis_name="c", subcore_axis_name="s", num_cores=NC, num_subcores=NS
    ),
    scratch_types=[
        pltpu.VMEM((B,), jnp.int32),
        pltpu.VMEM((B,), jnp.int32),
        pltpu.VMEM((B, H), jnp.float32),
        pltpu.VMEM((B, 16), jnp.float32),
        pltpu.VMEM((RB, H), jnp.float32),
        pltpu.VMEM((RB, 16), jnp.float32),
        pltpu.VMEM((RB, H), jnp.float32),
        pltpu.VMEM((RB, 16), jnp.float32),
        pltpu.VMEM_SHARED((NPAD, H), jnp.float32),
        pltpu.VMEM_SHARED((NPAD, 16), jnp.float32),
        pltpu.SemaphoreType.DMA,
    ],
)(_sc_body)


def _tc_body(xi, xu, mi_lo, mi_hi, mu_lo, mu_hi, wst, wnt_lo, wnt_hi, oi, ou):
    f32 = jnp.float32
    oi[...] = (
        jnp.dot(xi[...], wst[...], preferred_element_type=f32)
        + jnp.dot(mi_lo[0, 0], wnt_lo[...], preferred_element_type=f32)
        + jnp.dot(mi_hi[0, 0], wnt_hi[...], preferred_element_type=f32)
    )
    ou[...] = (
        jnp.dot(xu[...], wst[...], preferred_element_type=f32)
        + jnp.dot(mu_lo[0, 0], wnt_lo[...], preferred_element_type=f32)
        + jnp.dot(mu_hi[0, 0], wnt_hi[...], preferred_element_type=f32)
    )


ROWS_BLK = 1000
GRID = N // ROWS_BLK


def _mean_spec(d, c):
    return pl.BlockSpec(
        (1, 1, ROWS_BLK, H), lambda i, d=d, c=c: (d, c, i, 0)
    )


_tc_dense = pl.pallas_call(
    _tc_body,
    grid=(GRID,),
    in_specs=[
        pl.BlockSpec((ROWS_BLK, D), lambda i: (i, 0)),
        pl.BlockSpec((ROWS_BLK, D), lambda i: (i, 0)),
        _mean_spec(0, 0),
        _mean_spec(0, 1),
        _mean_spec(1, 0),
        _mean_spec(1, 1),
        pl.BlockSpec((D, D), lambda i: (0, 0)),
        pl.BlockSpec((H, D), lambda i: (0, 0)),
        pl.BlockSpec((H, D), lambda i: (0, 0)),
    ],
    out_specs=[
        pl.BlockSpec((ROWS_BLK, D), lambda i: (i, 0)),
        pl.BlockSpec((ROWS_BLK, D), lambda i: (i, 0)),
    ],
    out_shape=[
        jax.ShapeDtypeStruct((N, D), jnp.float32),
        jax.ShapeDtypeStruct((N, D), jnp.float32),
    ],
)


def kernel(user_feat, item_feat, edge_index_ui, edge_index_iu, W_self, W_neigh):
    pad = EPAD - E
    spad = jnp.arange(pad, dtype=jnp.int32) % N
    dpad = N + jnp.arange(pad, dtype=jnp.int32) % (NPAD - N)

    def prep(edge_index):
        src = edge_index[0].astype(jnp.int32)
        dst = edge_index[1].astype(jnp.int32)
        srcp = jnp.concatenate([src, spad])
        dstp = jnp.concatenate([dst, dpad])
        sidx2 = jnp.stack([srcp, srcp + N])
        return sidx2, dstp

    sidx_ui, didx_ui = prep(edge_index_ui)
    sidx_iu, didx_iu = prep(edge_index_iu)
    tab_ui = jnp.concatenate([user_feat[:, :H], user_feat[:, H:]], axis=0)
    tab_iu = jnp.concatenate([item_feat[:, :H], item_feat[:, H:]], axis=0)

    mean = _sc_agg(tab_ui, tab_iu, sidx_ui, didx_ui, sidx_iu, didx_iu)

    wst = W_self.T
    wnt = W_neigh.T
    rst_item, rst_user = _tc_dense(
        item_feat, user_feat, mean, mean, mean, mean, wst, wnt[:H], wnt[H:]
    )
    return (rst_user, rst_item)


# speedup vs baseline: 2.3671x; 2.3671x over previous
import functools

import jax
import jax.numpy as jnp
from jax import lax
from jax.experimental import pallas as pl
from jax.experimental.pallas import tpu as pltpu
from jax.experimental.pallas import tpu_sc as plsc

N = 10000
D = 256
HQ = 64
CW = 8
E = 160000
NC = 2
NS = 16
B = 128
NBLK = 80
EPT = B * NBLK
EPAD = EPT * NS
NPT = 640
NPAD = NPT * NS
RB = 128
NRB = NPT // RB


def _sc_body(tab_ui, tab_iu, sidx_ui, didx_ui, sidx_iu, didx_iu,
             ones8, zrow, zcnt,
             sum_out, cnt_out,
             sidx_v, didx_v, rows_v, ones_v, zero_v, zero16_v,
             acc_sh, cnt_sh, gsem):
    c = lax.axis_index("c")
    s = lax.axis_index("s")
    ebase = s * EPT
    rbase = s * NPT

    pltpu.sync_copy(ones8, ones_v)
    pltpu.sync_copy(zrow, zero_v)
    pltpu.sync_copy(zcnt, zero16_v)

    for d, (tab, sidx, didx) in enumerate(
        ((tab_ui, sidx_ui, didx_ui), (tab_iu, sidx_iu, didx_iu))
    ):
      for p in range(2):
        q = c * 2 + p
        for k in range(NRB):
            pltpu.sync_copy(zero_v, acc_sh.at[pl.ds(rbase + k * RB, RB)])
            if p == 0:
                pltpu.sync_copy(zero16_v, cnt_sh.at[pl.ds(rbase + k * RB, RB)])
        plsc.subcore_barrier()

        def _blk(b, carry):
            off = ebase + b * B
            pltpu.sync_copy(sidx.at[q, pl.ds(off, B)], sidx_v)
            pltpu.sync_copy(didx.at[pl.ds(off, B)], didx_v)
            pltpu.async_copy(tab.at[sidx_v], rows_v, gsem).wait()
            pltpu.sync_copy(rows_v, acc_sh.at[didx_v], add=True)
            if p == 0:
                pltpu.sync_copy(ones_v, cnt_sh.at[didx_v], add=True)
            return carry

        lax.fori_loop(0, NBLK, _blk, 0)
        plsc.subcore_barrier()

        pltpu.sync_copy(
            acc_sh.at[pl.ds(rbase, NPT)], sum_out.at[d, q, pl.ds(rbase, NPT)]
        )
        if p == 0:
            pltpu.sync_copy(
                cnt_sh.at[pl.ds(rbase, NPT)], cnt_out.at[d, pl.ds(rbase, NPT)]
            )


_sc_agg = functools.partial(
    pl.kernel,
    out_type=(
        jax.ShapeDtypeStruct((2, 4, NPAD, HQ), jnp.float32),
        jax.ShapeDtypeStruct((2, NPAD, CW), jnp.float32),
    ),
    mesh=plsc.VectorSubcoreMesh(
        core_axis_name="c", subcore_axis_name="s", num_cores=NC, num_subcores=NS
    ),
    scratch_types=[
        pltpu.VMEM((B,), jnp.int32),
        pltpu.VMEM((B,), jnp.int32),
        pltpu.VMEM((B, HQ), jnp.float32),
        pltpu.VMEM((B, CW), jnp.float32),
        pltpu.VMEM((RB, HQ), jnp.float32),
        pltpu.VMEM((RB, CW), jnp.float32),
        pltpu.VMEM_SHARED((NPAD, HQ), jnp.float32),
        pltpu.VMEM_SHARED((NPAD, CW), jnp.float32),
        pltpu.SemaphoreType.DMA,
    ],
    compiler_params=pltpu.CompilerParams(use_tc_tiling_on_sc=False),
)(_sc_body)


def _tc_body(xi, xu, si0, si1, si2, si3, su0, su1, su2, su3, ci, cu,
             wst, wn0, wn1, wn2, wn3, oi, ou):
    f32 = jnp.float32

    def one(x, sq, cnt, out):
        acc = jnp.dot(x[...], wst[...], preferred_element_type=f32)
        neigh = sum(
            jnp.dot(sq[j][0, 0], (wn0, wn1, wn2, wn3)[j][...],
                    preferred_element_type=f32)
            for j in range(4)
        )
        scale = 1.0 / jnp.maximum(cnt[0][:, 0:1], 1.0)
        out[...] = acc + neigh * scale

    one(xi, (si0, si1, si2, si3), ci, oi)
    one(xu, (su0, su1, su2, su3), cu, ou)


ROWS_BLK = 1000
GRID = N // ROWS_BLK


def _sum_spec(d, q):
    return pl.BlockSpec(
        (1, 1, ROWS_BLK, HQ), lambda i, d=d, q=q: (d, q, i, 0)
    )


def _cnt_spec(d):
    return pl.BlockSpec((1, ROWS_BLK, CW), lambda i, d=d: (d, i, 0))


_tc_dense = pl.pallas_call(
    _tc_body,
    grid=(GRID,),
    in_specs=[
        pl.BlockSpec((ROWS_BLK, D), lambda i: (i, 0)),
        pl.BlockSpec((ROWS_BLK, D), lambda i: (i, 0)),
        _sum_spec(0, 0), _sum_spec(0, 1), _sum_spec(0, 2), _sum_spec(0, 3),
        _sum_spec(1, 0), _sum_spec(1, 1), _sum_spec(1, 2), _sum_spec(1, 3),
        _cnt_spec(0), _cnt_spec(1),
        pl.BlockSpec((D, D), lambda i: (0, 0)),
        pl.BlockSpec((HQ, D), lambda i: (0, 0)),
        pl.BlockSpec((HQ, D), lambda i: (0, 0)),
        pl.BlockSpec((HQ, D), lambda i: (0, 0)),
        pl.BlockSpec((HQ, D), lambda i: (0, 0)),
    ],
    out_specs=[
        pl.BlockSpec((ROWS_BLK, D), lambda i: (i, 0)),
        pl.BlockSpec((ROWS_BLK, D), lambda i: (i, 0)),
    ],
    out_shape=[
        jax.ShapeDtypeStruct((N, D), jnp.float32),
        jax.ShapeDtypeStruct((N, D), jnp.float32),
    ],
)


def kernel(user_feat, item_feat, edge_index_ui, edge_index_iu, W_self, W_neigh):
    pad = EPAD - E
    spad = jnp.arange(pad, dtype=jnp.int32) % N
    dpad = N + jnp.arange(pad, dtype=jnp.int32) % (NPAD - N)

    def prep(edge_index):
        src = edge_index[0].astype(jnp.int32)
        dst = edge_index[1].astype(jnp.int32)
        srcp = jnp.concatenate([src, spad])
        dstp = jnp.concatenate([dst, dpad])
        sidx2 = jnp.stack([srcp, srcp + N, srcp + 2 * N, srcp + 3 * N])
        return sidx2, dstp

    sidx_ui, didx_ui = prep(edge_index_ui)
    sidx_iu, didx_iu = prep(edge_index_iu)

    def split_cols(f):
        return jnp.concatenate(
            [f[:, q * HQ:(q + 1) * HQ] for q in range(4)], axis=0
        )

    tab_ui = split_cols(user_feat)
    tab_iu = split_cols(item_feat)

    ones8 = jnp.ones((B, CW), jnp.float32)
    zrow = jnp.zeros((RB, HQ), jnp.float32)
    zcnt = jnp.zeros((RB, CW), jnp.float32)

    sums, cnts = _sc_agg(
        tab_ui, tab_iu, sidx_ui, didx_ui, sidx_iu, didx_iu, ones8, zrow, zcnt
    )

    wst = W_self.T
    wnt = W_neigh.T
    wnq = [wnt[q * HQ:(q + 1) * HQ] for q in range(4)]
    rst_item, rst_user = _tc_dense(
        item_feat, user_feat,
        sums, sums, sums, sums, sums, sums, sums, sums,
        cnts, cnts,
        wst, *wnq,
    )
    return (rst_user, rst_item)
